# Initial kernel scaffold; baseline (speedup 1.0000x reference)
import jax
import jax.numpy as jnp
from jax.experimental import pallas as pl


def kernel(x, edge_index, batch, W1, b1, W2, b2, W_iou, b_iou, W_iop, b_iop):
    raise NotImplementedError("write your pallas kernel here")


# speedup vs baseline: 18.3164x; 18.3164x over previous
import functools

import jax
import jax.numpy as jnp
from jax import lax
from jax.experimental import pallas as pl
from jax.experimental.pallas import tpu as pltpu
from jax.experimental.pallas import tpu_sc as plsc

N = 10000
E = 320000
D = 128
G = 64
NW = 32
EPW = E // NW
CH = 125
NCH = EPW // CH
RPT = 640
RA = 632
RL = N - 15 * RA

_mesh = plsc.VectorSubcoreMesh(core_axis_name="c", subcore_axis_name="s")


@functools.partial(
    pl.kernel,
    mesh=_mesh,
    out_type=jax.ShapeDtypeStruct((2 * 16 * RPT,), jnp.float32),
    scratch_types=[
        pltpu.VMEM((NCH, CH), jnp.int32),
        pltpu.VMEM((RPT,), jnp.float32),
        pltpu.VMEM_SHARED((16 * RPT,), jnp.float32),
    ],
)
def _deg_sc(dst_hbm, out_hbm, idx_v, buf_v, acc_sh):
    c = lax.axis_index("c")
    s = lax.axis_index("s")
    wid = c * 16 + s

    def _fill(i, val):
        buf_v[pl.ds(i * 16, 16)] = jnp.full((16,), val, jnp.float32)

    drow0 = pl.multiple_of(s * RPT, 8)
    dorow0 = pl.multiple_of(c * 16 * RPT + s * RPT, 8)
    lax.fori_loop(0, RPT // 16, lambda i, _: (_fill(i, 0.0), 0)[1], 0)
    pltpu.sync_copy(buf_v, acc_sh.at[pl.ds(drow0, RPT)])
    lax.fori_loop(0, 8, lambda i, _: (_fill(i, 1.0), 0)[1], 0)
    pltpu.sync_copy(dst_hbm.at[wid], idx_v)
    plsc.subcore_barrier()

    def _body(j, _):
        pltpu.sync_copy(buf_v.at[pl.ds(0, CH)], acc_sh.at[idx_v.at[j]],
                        add=True)
        return 0

    lax.fori_loop(0, NCH, _body, 0)
    plsc.subcore_barrier()
    pltpu.sync_copy(acc_sh.at[pl.ds(drow0, RPT)],
                    out_hbm.at[pl.ds(dorow0, RPT)])


@functools.partial(
    pl.kernel,
    mesh=_mesh,
    out_type=jax.ShapeDtypeStruct((2 * N, D), jnp.float32),
    scratch_types=[
        pltpu.VMEM((NCH, CH), jnp.int32),
        pltpu.VMEM((NCH, CH), jnp.int32),
        pltpu.VMEM((CH, D), jnp.float32),
        pltpu.VMEM_SHARED((N, D), jnp.float32),
        pltpu.SemaphoreType.DMA,
    ],
)
def _agg_sc(g_hbm, src_hbm, dst_hbm, out_hbm, src_v, dst_v, rows_v, acc_sh,
            sem):
    c = lax.axis_index("c")
    s = lax.axis_index("s")
    wid = c * 16 + s
    row0 = pl.multiple_of(s * RA, 8)
    orow0 = pl.multiple_of(c * N + s * RA, 8)

    @pl.when(s < 15)
    def _():
        pltpu.sync_copy(g_hbm.at[pl.ds(row0, RA)], acc_sh.at[pl.ds(row0, RA)])

    @pl.when(s == 15)
    def _():
        pltpu.sync_copy(g_hbm.at[pl.ds(row0, RL)], acc_sh.at[pl.ds(row0, RL)])

    pltpu.sync_copy(src_hbm.at[wid], src_v)
    pltpu.sync_copy(dst_hbm.at[wid], dst_v)
    plsc.subcore_barrier()

    def _body(j, _):
        pltpu.async_copy(g_hbm.at[src_v.at[j]], rows_v, sem).wait()
        pltpu.sync_copy(rows_v, acc_sh.at[dst_v.at[j]], add=True)
        return 0

    lax.fori_loop(0, NCH, _body, 0)
    plsc.subcore_barrier()

    @pl.when(s < 15)
    def _():
        pltpu.sync_copy(acc_sh.at[pl.ds(row0, RA)],
                        out_hbm.at[pl.ds(orow0, RA)])

    @pl.when(s == 15)
    def _():
        pltpu.sync_copy(acc_sh.at[pl.ds(row0, RL)],
                        out_hbm.at[pl.ds(orow0, RL)])


_BM = 2000


def _lin1_body(d0_ref, d1_ref, x_ref, w_ref, g_ref, dis_ref):
    deg = d0_ref[...] + d1_ref[...] + 1.0
    dis = lax.rsqrt(deg)
    hw = jnp.dot(x_ref[...], w_ref[...], preferred_element_type=jnp.float32)
    g_ref[...] = hw * dis
    dis_ref[...] = dis


def _lin2_body(p0_ref, p1_ref, g_ref, dis_ref, b_ref, w_ref, out_ref):
    dis = dis_ref[...]
    h = (p0_ref[...] + p1_ref[...] - g_ref[...]) * dis + b_ref[...]
    h = jnp.maximum(h, 0.0)
    out_ref[...] = jnp.dot(h, w_ref[...],
                           preferred_element_type=jnp.float32) * dis


def _relu2_body(p0_ref, p1_ref, g_ref, dis_ref, b_ref, out_ref):
    h = (p0_ref[...] + p1_ref[...] - g_ref[...]) * dis_ref[...] + b_ref[...]
    out_ref[...] = jnp.maximum(h, 0.0)


def _pool_body(h_ref, batch_ref, w_ref, bc_ref, out_ref, pool_ref):
    neg = jnp.float32(-jnp.inf)

    def body(gi, _):
        m = batch_ref[...] == gi
        v = jnp.max(jnp.where(m, h_ref[...], neg), axis=0, keepdims=True)
        pool_ref[pl.ds(gi, 1), :] = v
        return 0

    lax.fori_loop(0, G, body, 0)
    out_ref[...] = jnp.dot(pool_ref[...], w_ref[...],
                           preferred_element_type=jnp.float32) + bc_ref[...]


def _row_blocked(width):
    return pl.BlockSpec((_BM, width), lambda i: (i, 0))


def _whole(shape):
    return pl.BlockSpec(shape, lambda *a: tuple(0 for _ in shape))


def kernel(x, edge_index, batch, W1, b1, W2, b2, W_iou, b_iou, W_iop, b_iop):
    src3 = edge_index[0].reshape(NW, NCH, CH)
    dst3 = edge_index[1].reshape(NW, NCH, CH)

    degp = _deg_sc(dst3)
    deg0 = degp[0:N].reshape(N, 1)
    deg1 = degp[16 * RPT:16 * RPT + N].reshape(N, 1)

    grid = (N // _BM,)
    g1, dis = pl.pallas_call(
        _lin1_body,
        grid=grid,
        in_specs=[_row_blocked(1), _row_blocked(1), _row_blocked(D),
                  _whole((D, D))],
        out_specs=[_row_blocked(D), _row_blocked(1)],
        out_shape=[jax.ShapeDtypeStruct((N, D), jnp.float32),
                   jax.ShapeDtypeStruct((N, 1), jnp.float32)],
    )(deg0, deg1, x, W1)

    p = _agg_sc(g1, src3, dst3)

    g2 = pl.pallas_call(
        _lin2_body,
        grid=grid,
        in_specs=[_row_blocked(D), _row_blocked(D), _row_blocked(D),
                  _row_blocked(1), _whole((1, D)), _whole((D, D))],
        out_specs=_row_blocked(D),
        out_shape=jax.ShapeDtypeStruct((N, D), jnp.float32),
    )(p[:N], p[N:], g1, dis, b1.reshape(1, D), W2)

    p2 = _agg_sc(g2, src3, dst3)

    h2 = pl.pallas_call(
        _relu2_body,
        grid=grid,
        in_specs=[_row_blocked(D), _row_blocked(D), _row_blocked(D),
                  _row_blocked(1), _whole((1, D))],
        out_specs=_row_blocked(D),
        out_shape=jax.ShapeDtypeStruct((N, D), jnp.float32),
    )(p2[:N], p2[N:], g2, dis, b2.reshape(1, D))

    w_cat = jnp.concatenate([W_iou, W_iop], axis=1)
    b_cat = jnp.concatenate([b_iou, b_iop]).reshape(1, 2)
    out = pl.pallas_call(
        _pool_body,
        in_specs=[_whole((N, D)), _whole((N, 1)), _whole((D, 2)),
                  _whole((1, 2))],
        out_specs=_whole((G, 2)),
        out_shape=jax.ShapeDtypeStruct((G, 2), jnp.float32),
        scratch_shapes=[pltpu.VMEM((G, D), jnp.float32)],
    )(h2, batch.reshape(N, 1), w_cat, b_cat)
    return out


# speedup vs baseline: 25.0556x; 1.3679x over previous
import functools

import jax
import jax.numpy as jnp
from jax import lax
from jax.experimental import pallas as pl
from jax.experimental.pallas import tpu as pltpu
from jax.experimental.pallas import tpu_sc as plsc

N = 10000
E = 320000
D = 128
G = 64
NW = 32
EPW = E // NW
CH = 100
NCH = EPW // CH
RPT = 640
RA = 632
RL = N - 15 * RA

_mesh = plsc.VectorSubcoreMesh(core_axis_name="c", subcore_axis_name="s")


@functools.partial(
    pl.kernel,
    mesh=_mesh,
    out_type=jax.ShapeDtypeStruct((2 * 16 * RPT,), jnp.float32),
    scratch_types=[
        pltpu.VMEM((NCH, CH), jnp.int32),
        pltpu.VMEM((RPT,), jnp.float32),
        pltpu.VMEM_SHARED((16 * RPT,), jnp.float32),
    ],
)
def _deg_sc(dst_hbm, out_hbm, idx_v, buf_v, acc_sh):
    c = lax.axis_index("c")
    s = lax.axis_index("s")
    wid = c * 16 + s

    def _fill(i, val):
        buf_v[pl.ds(i * 16, 16)] = jnp.full((16,), val, jnp.float32)

    drow0 = pl.multiple_of(s * RPT, 8)
    dorow0 = pl.multiple_of(c * 16 * RPT + s * RPT, 8)
    lax.fori_loop(0, RPT // 16, lambda i, _: (_fill(i, 0.0), 0)[1], 0)
    pltpu.sync_copy(buf_v, acc_sh.at[pl.ds(drow0, RPT)])
    lax.fori_loop(0, 8, lambda i, _: (_fill(i, 1.0), 0)[1], 0)
    pltpu.sync_copy(dst_hbm.at[wid], idx_v)
    plsc.subcore_barrier()

    def _body(j, _):
        pltpu.sync_copy(buf_v.at[pl.ds(0, CH)], acc_sh.at[idx_v.at[j]],
                        add=True)
        return 0

    lax.fori_loop(0, NCH, _body, 0)
    plsc.subcore_barrier()
    pltpu.sync_copy(acc_sh.at[pl.ds(drow0, RPT)],
                    out_hbm.at[pl.ds(dorow0, RPT)])


@functools.partial(
    pl.kernel,
    mesh=_mesh,
    out_type=jax.ShapeDtypeStruct((2 * N, D), jnp.float32),
    scratch_types=[
        pltpu.VMEM((2, CH), jnp.int32),
        pltpu.VMEM((2, CH), jnp.int32),
        pltpu.VMEM((2, CH), jnp.int32),
        pltpu.VMEM((2, CH), jnp.int32),
        pltpu.VMEM((CH, D), jnp.float32),
        pltpu.VMEM((CH, D), jnp.float32),
        pltpu.VMEM_SHARED((N, D), jnp.float32),
        pltpu.SemaphoreType.DMA,
        pltpu.SemaphoreType.DMA,
        pltpu.SemaphoreType.DMA,
        pltpu.SemaphoreType.DMA,
        pltpu.SemaphoreType.DMA,
        pltpu.SemaphoreType.DMA,
    ],
)
def _agg_sc(g_hbm, eidx_hbm, out_hbm, b0, b1, b2, b3, r0, r1,
            acc_sh, si0, si1, si2, si3, sr0, sr1):
    c = lax.axis_index("c")
    s = lax.axis_index("s")
    wid = c * 16 + s
    row0 = pl.multiple_of(s * RA, 8)
    orow0 = pl.multiple_of(c * N + s * RA, 8)
    bufs = (b0, b1, b2, b3)
    isems = (si0, si1, si2, si3)
    rows = (r0, r1)
    rsems = (sr0, sr1)

    def il(j, k):
        pltpu.async_copy(eidx_hbm.at[wid, j], bufs[k], isems[k])

    def wi(j, k):
        pltpu.make_async_copy(eidx_hbm.at[wid, j], bufs[k], isems[k]).wait()

    def ga(k, p):
        pltpu.async_copy(g_hbm.at[bufs[k].at[0]], rows[p], rsems[p])

    def wr(k, p):
        pltpu.make_async_copy(g_hbm.at[bufs[k].at[0]], rows[p],
                              rsems[p]).wait()

    def sc(k, p):
        pltpu.sync_copy(rows[p], acc_sh.at[bufs[k].at[1]], add=True)


    @pl.when(s < 15)
    def _():
        pltpu.sync_copy(g_hbm.at[pl.ds(row0, RA)], acc_sh.at[pl.ds(row0, RA)])

    @pl.when(s == 15)
    def _():
        pltpu.sync_copy(g_hbm.at[pl.ds(row0, RL)], acc_sh.at[pl.ds(row0, RL)])

    il(0, 0)
    il(1, 1)
    il(2, 2)
    il(3, 3)
    plsc.subcore_barrier()
    wi(0, 0)
    ga(0, 0)

    NB = NCH // 4

    def _body(bb, _):
        c0 = bb * 4
        last = bb >= NB - 1
        wi(c0 + 1, 1)
        ga(1, 1)
        wr(0, 0)
        sc(0, 0)

        @pl.when(~last)
        def _():
            il(c0 + 4, 0)

        wi(c0 + 2, 2)
        ga(2, 0)
        wr(1, 1)
        sc(1, 1)

        @pl.when(~last)
        def _():
            il(c0 + 5, 1)

        wi(c0 + 3, 3)
        ga(3, 1)
        wr(2, 0)
        sc(2, 0)

        @pl.when(~last)
        def _():
            il(c0 + 6, 2)
            wi(c0 + 4, 0)
            ga(0, 0)

        wr(3, 1)
        sc(3, 1)

        @pl.when(~last)
        def _():
            il(c0 + 7, 3)

        return 0

    lax.fori_loop(0, NB, _body, 0)
    plsc.subcore_barrier()

    @pl.when(s < 15)
    def _():
        pltpu.sync_copy(acc_sh.at[pl.ds(row0, RA)],
                        out_hbm.at[pl.ds(orow0, RA)])

    @pl.when(s == 15)
    def _():
        pltpu.sync_copy(acc_sh.at[pl.ds(row0, RL)],
                        out_hbm.at[pl.ds(orow0, RL)])


_BM = 2000


def _lin1_body(d0_ref, d1_ref, x_ref, w_ref, g_ref, dis_ref):
    deg = d0_ref[...] + d1_ref[...] + 1.0
    dis = lax.rsqrt(deg)
    hw = jnp.dot(x_ref[...], w_ref[...], preferred_element_type=jnp.float32)
    g_ref[...] = hw * dis
    dis_ref[...] = dis


def _lin2_body(p0_ref, p1_ref, g_ref, dis_ref, b_ref, w_ref, out_ref):
    dis = dis_ref[...]
    h = (p0_ref[...] + p1_ref[...] - g_ref[...]) * dis + b_ref[...]
    h = jnp.maximum(h, 0.0)
    out_ref[...] = jnp.dot(h, w_ref[...],
                           preferred_element_type=jnp.float32) * dis


def _pool_body(p0_ref, p1_ref, g_ref, dis_ref, b_ref, batch_ref, w_ref,
               bc_ref, out_ref, h_ref, pool_ref):
    h = (p0_ref[...] + p1_ref[...] - g_ref[...]) * dis_ref[...] + b_ref[...]
    h_ref[...] = jnp.maximum(h, 0.0)
    neg = jnp.float32(-jnp.inf)

    def body(gi, _):
        m = batch_ref[...] == gi
        v = jnp.max(jnp.where(m, h_ref[...], neg), axis=0, keepdims=True)
        pool_ref[pl.ds(gi, 1), :] = v
        return 0

    lax.fori_loop(0, G, body, 0)
    out_ref[...] = jnp.dot(pool_ref[...], w_ref[...],
                           preferred_element_type=jnp.float32) + bc_ref[...]


def _row_blocked(width):
    return pl.BlockSpec((_BM, width), lambda i: (i, 0))


def _whole(shape):
    return pl.BlockSpec(shape, lambda *a: tuple(0 for _ in shape))


def kernel(x, edge_index, batch, W1, b1, W2, b2, W_iou, b_iou, W_iop, b_iop):
    src3 = edge_index[0].reshape(NW, NCH, CH)
    dst3 = edge_index[1].reshape(NW, NCH, CH)
    eidx = jnp.stack([src3, dst3], axis=2)

    degp = _deg_sc(dst3)
    deg0 = degp[0:N].reshape(N, 1)
    deg1 = degp[16 * RPT:16 * RPT + N].reshape(N, 1)

    grid = (N // _BM,)
    g1, dis = pl.pallas_call(
        _lin1_body,
        grid=grid,
        in_specs=[_row_blocked(1), _row_blocked(1), _row_blocked(D),
                  _whole((D, D))],
        out_specs=[_row_blocked(D), _row_blocked(1)],
        out_shape=[jax.ShapeDtypeStruct((N, D), jnp.float32),
                   jax.ShapeDtypeStruct((N, 1), jnp.float32)],
    )(deg0, deg1, x, W1)

    p = _agg_sc(g1, eidx)

    nb = N // _BM
    p_lo = pl.BlockSpec((_BM, D), lambda i: (i, 0))
    p_hi = pl.BlockSpec((_BM, D), lambda i: (i + nb, 0))
    g2 = pl.pallas_call(
        _lin2_body,
        grid=grid,
        in_specs=[p_lo, p_hi, _row_blocked(D),
                  _row_blocked(1), _whole((1, D)), _whole((D, D))],
        out_specs=_row_blocked(D),
        out_shape=jax.ShapeDtypeStruct((N, D), jnp.float32),
    )(p, p, g1, dis, b1.reshape(1, D), W2)

    p2 = _agg_sc(g2, eidx)

    w_cat = jnp.concatenate([W_iou, W_iop], axis=1)
    b_cat = jnp.concatenate([b_iou, b_iop]).reshape(1, 2)
    w_lo = pl.BlockSpec((N, D), lambda i: (0, 0))
    w_hi = pl.BlockSpec((N, D), lambda i: (1, 0))
    out = pl.pallas_call(
        _pool_body,
        grid=(1,),
        in_specs=[w_lo, w_hi, _whole((N, D)), _whole((N, 1)),
                  _whole((1, D)), _whole((N, 1)), _whole((D, 2)),
                  _whole((1, 2))],
        out_specs=_whole((G, 2)),
        out_shape=jax.ShapeDtypeStruct((G, 2), jnp.float32),
        scratch_shapes=[pltpu.VMEM((N, D), jnp.float32),
                        pltpu.VMEM((G, D), jnp.float32)],
    )(p2, p2, g2, dis, b2.reshape(1, D), batch.reshape(N, 1), w_cat, b_cat)
    return out


# speedup vs baseline: 31.4866x; 1.2567x over previous
import functools

import jax
import jax.numpy as jnp
from jax import lax
from jax.experimental import pallas as pl
from jax.experimental.pallas import tpu as pltpu
from jax.experimental.pallas import tpu_sc as plsc

N = 10000
E = 320000
D = 128
G = 64
NW = 32
EPW = E // NW
CH = 100
NCH = EPW // CH
RPT = 640
RA = 632
RL = N - 15 * RA

_mesh = plsc.VectorSubcoreMesh(core_axis_name="c", subcore_axis_name="s")


@functools.partial(
    pl.kernel,
    mesh=_mesh,
    out_type=jax.ShapeDtypeStruct((2 * 16 * RPT,), jnp.float32),
    scratch_types=[
        pltpu.VMEM((NCH, CH), jnp.int32),
        pltpu.VMEM((RPT,), jnp.float32),
        pltpu.VMEM_SHARED((16 * RPT,), jnp.float32),
    ],
)
def _deg_sc(dst_hbm, out_hbm, idx_v, buf_v, acc_sh):
    c = lax.axis_index("c")
    s = lax.axis_index("s")
    wid = c * 16 + s

    def _fill(i, val):
        buf_v[pl.ds(i * 16, 16)] = jnp.full((16,), val, jnp.float32)

    drow0 = pl.multiple_of(s * RPT, 8)
    dorow0 = pl.multiple_of(c * 16 * RPT + s * RPT, 8)
    lax.fori_loop(0, RPT // 16, lambda i, _: (_fill(i, 0.0), 0)[1], 0)
    pltpu.sync_copy(buf_v, acc_sh.at[pl.ds(drow0, RPT)])
    lax.fori_loop(0, 8, lambda i, _: (_fill(i, 1.0), 0)[1], 0)
    pltpu.sync_copy(dst_hbm.at[wid], idx_v)
    plsc.subcore_barrier()

    def _body(j, _):
        pltpu.sync_copy(buf_v.at[pl.ds(0, CH)], acc_sh.at[idx_v.at[j]],
                        add=True)
        return 0

    lax.fori_loop(0, NCH, _body, 0)
    plsc.subcore_barrier()
    pltpu.sync_copy(acc_sh.at[pl.ds(drow0, RPT)],
                    out_hbm.at[pl.ds(dorow0, RPT)])


@functools.partial(
    pl.kernel,
    mesh=_mesh,
    out_type=jax.ShapeDtypeStruct((2 * N, D), jnp.float32),
    scratch_types=[
        pltpu.VMEM((2, CH), jnp.int32),
        pltpu.VMEM((2, CH), jnp.int32),
        pltpu.VMEM((2, CH), jnp.int32),
        pltpu.VMEM((2, CH), jnp.int32),
        pltpu.VMEM((CH, D), jnp.float32),
        pltpu.VMEM((CH, D), jnp.float32),
        pltpu.VMEM_SHARED((N, D), jnp.float32),
        pltpu.SemaphoreType.DMA,
        pltpu.SemaphoreType.DMA,
        pltpu.SemaphoreType.DMA,
        pltpu.SemaphoreType.DMA,
        pltpu.SemaphoreType.DMA,
        pltpu.SemaphoreType.DMA,
    ],
)
def _agg_sc(g_hbm, eidx_hbm, out_hbm, b0, b1, b2, b3, r0, r1,
            acc_sh, si0, si1, si2, si3, sr0, sr1):
    c = lax.axis_index("c")
    s = lax.axis_index("s")
    wid = c * 16 + s
    row0 = pl.multiple_of(s * RA, 8)
    orow0 = pl.multiple_of(c * N + s * RA, 8)
    bufs = (b0, b1, b2, b3)
    isems = (si0, si1, si2, si3)
    rows = (r0, r1)
    rsems = (sr0, sr1)

    def il(j, k):
        pltpu.async_copy(eidx_hbm.at[wid, j], bufs[k], isems[k])

    def wi(j, k):
        pltpu.make_async_copy(eidx_hbm.at[wid, j], bufs[k], isems[k]).wait()

    def ga(k, p):
        pltpu.async_copy(g_hbm.at[bufs[k].at[0]], rows[p], rsems[p])

    def wr(k, p):
        pltpu.make_async_copy(g_hbm.at[bufs[k].at[0]], rows[p],
                              rsems[p]).wait()

    def sc(k, p):
        pltpu.sync_copy(rows[p], acc_sh.at[bufs[k].at[1]], add=True)


    @pl.when(s < 15)
    def _():
        pltpu.sync_copy(g_hbm.at[pl.ds(row0, RA)], acc_sh.at[pl.ds(row0, RA)])

    @pl.when(s == 15)
    def _():
        pltpu.sync_copy(g_hbm.at[pl.ds(row0, RL)], acc_sh.at[pl.ds(row0, RL)])

    il(0, 0)
    il(1, 1)
    il(2, 2)
    il(3, 3)
    plsc.subcore_barrier()
    wi(0, 0)
    ga(0, 0)

    NB = NCH // 4

    def _body(bb, _):
        c0 = bb * 4
        last = bb >= NB - 1
        wi(c0 + 1, 1)
        ga(1, 1)
        wr(0, 0)
        sc(0, 0)

        @pl.when(~last)
        def _():
            il(c0 + 4, 0)

        wi(c0 + 2, 2)
        ga(2, 0)
        wr(1, 1)
        sc(1, 1)

        @pl.when(~last)
        def _():
            il(c0 + 5, 1)

        wi(c0 + 3, 3)
        ga(3, 1)
        wr(2, 0)
        sc(2, 0)

        @pl.when(~last)
        def _():
            il(c0 + 6, 2)
            wi(c0 + 4, 0)
            ga(0, 0)

        wr(3, 1)
        sc(3, 1)

        @pl.when(~last)
        def _():
            il(c0 + 7, 3)

        return 0

    lax.fori_loop(0, NB, _body, 0)
    plsc.subcore_barrier()

    @pl.when(s < 15)
    def _():
        pltpu.sync_copy(acc_sh.at[pl.ds(row0, RA)],
                        out_hbm.at[pl.ds(orow0, RA)])

    @pl.when(s == 15)
    def _():
        pltpu.sync_copy(acc_sh.at[pl.ds(row0, RL)],
                        out_hbm.at[pl.ds(orow0, RL)])


_BM = 2000


def _lin1_body(d0_ref, d1_ref, x_ref, w_ref, g_ref, dis_ref):
    deg = d0_ref[...] + d1_ref[...] + 1.0
    dis = lax.rsqrt(deg)
    hw = jnp.dot(x_ref[...], w_ref[...], preferred_element_type=jnp.float32)
    g_ref[...] = hw * dis
    dis_ref[...] = dis


def _lin2_body(p0_ref, p1_ref, g_ref, dis_ref, b_ref, w_ref, out_ref):
    dis = dis_ref[...]
    h = (p0_ref[...] + p1_ref[...] - g_ref[...]) * dis + b_ref[...]
    h = jnp.maximum(h, 0.0)
    out_ref[...] = jnp.dot(h, w_ref[...],
                           preferred_element_type=jnp.float32) * dis


def _pool_body(offs_ref, p0_ref, p1_ref, g_ref, dis_ref, b_ref, w_ref,
               bc_ref, out_ref, h_ref, pool_ref):
    h = (p0_ref[...] + p1_ref[...] - g_ref[...]) * dis_ref[...] + b_ref[...]
    h_ref[...] = jnp.maximum(h, 0.0)
    neg = jnp.float32(-jnp.inf)
    iota8 = lax.broadcasted_iota(jnp.int32, (8, 1), 0)

    def body(gi, _):
        s0 = offs_ref[gi]
        s1 = offs_ref[gi + 1]

        def inner(b, acc):
            ri = b * 8 + iota8
            m = (ri >= s0) & (ri < s1)
            rows = h_ref[pl.ds(b * 8, 8), :]
            return jnp.maximum(acc, jnp.where(m, rows, neg))

        acc = lax.fori_loop(s0 // 8, (s1 + 7) // 8,
                            inner, jnp.full((8, D), neg, jnp.float32))
        pool_ref[pl.ds(gi, 1), :] = jnp.max(acc, axis=0, keepdims=True)
        return 0

    lax.fori_loop(0, G, body, 0)
    out_ref[...] = jnp.dot(pool_ref[...], w_ref[...],
                           preferred_element_type=jnp.float32) + bc_ref[...]


def _row_blocked(width):
    return pl.BlockSpec((_BM, width), lambda i: (i, 0))


def _whole(shape):
    return pl.BlockSpec(shape, lambda *a: tuple(0 for _ in shape))


def kernel(x, edge_index, batch, W1, b1, W2, b2, W_iou, b_iou, W_iop, b_iop):
    src3 = edge_index[0].reshape(NW, NCH, CH)
    dst3 = edge_index[1].reshape(NW, NCH, CH)
    eidx = jnp.stack([src3, dst3], axis=2)

    degp = _deg_sc(dst3)
    deg0 = degp[0:N].reshape(N, 1)
    deg1 = degp[16 * RPT:16 * RPT + N].reshape(N, 1)

    grid = (N // _BM,)
    g1, dis = pl.pallas_call(
        _lin1_body,
        grid=grid,
        in_specs=[_row_blocked(1), _row_blocked(1), _row_blocked(D),
                  _whole((D, D))],
        out_specs=[_row_blocked(D), _row_blocked(1)],
        out_shape=[jax.ShapeDtypeStruct((N, D), jnp.float32),
                   jax.ShapeDtypeStruct((N, 1), jnp.float32)],
    )(deg0, deg1, x, W1)

    p = _agg_sc(g1, eidx)

    nb = N // _BM
    p_lo = pl.BlockSpec((_BM, D), lambda i: (i, 0))
    p_hi = pl.BlockSpec((_BM, D), lambda i: (i + nb, 0))
    g2 = pl.pallas_call(
        _lin2_body,
        grid=grid,
        in_specs=[p_lo, p_hi, _row_blocked(D),
                  _row_blocked(1), _whole((1, D)), _whole((D, D))],
        out_specs=_row_blocked(D),
        out_shape=jax.ShapeDtypeStruct((N, D), jnp.float32),
    )(p, p, g1, dis, b1.reshape(1, D), W2)

    p2 = _agg_sc(g2, eidx)

    w_cat = jnp.concatenate([W_iou, W_iop], axis=1)
    b_cat = jnp.concatenate([b_iou, b_iop]).reshape(1, 2)
    offs = jnp.searchsorted(batch, jnp.arange(G + 1, dtype=jnp.int32),
                            side="left").astype(jnp.int32)
    w_lo = pl.BlockSpec((N, D), lambda i: (0, 0))
    w_hi = pl.BlockSpec((N, D), lambda i: (1, 0))
    out = pl.pallas_call(
        _pool_body,
        grid=(1,),
        in_specs=[pl.BlockSpec(memory_space=pltpu.SMEM),
                  w_lo, w_hi, _whole((N, D)), _whole((N, 1)),
                  _whole((1, D)), _whole((D, 2)), _whole((1, 2))],
        out_specs=_whole((G, 2)),
        out_shape=jax.ShapeDtypeStruct((G, 2), jnp.float32),
        scratch_shapes=[pltpu.VMEM((N, D), jnp.float32),
                        pltpu.VMEM((G, D), jnp.float32)],
    )(offs, p2, p2, g2, dis, b2.reshape(1, D), w_cat, b_cat)
    return out


# speedup vs baseline: 32.6642x; 1.0374x over previous
import functools

import jax
import jax.numpy as jnp
from jax import lax
from jax.experimental import pallas as pl
from jax.experimental.pallas import tpu as pltpu
from jax.experimental.pallas import tpu_sc as plsc

N = 10000
E = 320000
D = 128
G = 64
NW = 32
EPW = E // NW
CH = 125
NCH = EPW // CH
RPT = 640
RA = 632
RL = N - 15 * RA

_mesh = plsc.VectorSubcoreMesh(core_axis_name="c", subcore_axis_name="s")


@functools.partial(
    pl.kernel,
    mesh=_mesh,
    out_type=jax.ShapeDtypeStruct((2 * 16 * RPT,), jnp.float32),
    scratch_types=[
        pltpu.VMEM((NCH, CH), jnp.int32),
        pltpu.VMEM((RPT,), jnp.float32),
        pltpu.VMEM_SHARED((16 * RPT,), jnp.float32),
    ],
)
def _deg_sc(dst_hbm, out_hbm, idx_v, buf_v, acc_sh):
    c = lax.axis_index("c")
    s = lax.axis_index("s")
    wid = c * 16 + s

    def _fill(i, val):
        buf_v[pl.ds(i * 16, 16)] = jnp.full((16,), val, jnp.float32)

    drow0 = pl.multiple_of(s * RPT, 8)
    dorow0 = pl.multiple_of(c * 16 * RPT + s * RPT, 8)
    lax.fori_loop(0, RPT // 16, lambda i, _: (_fill(i, 0.0), 0)[1], 0)
    pltpu.sync_copy(buf_v, acc_sh.at[pl.ds(drow0, RPT)])
    lax.fori_loop(0, 8, lambda i, _: (_fill(i, 1.0), 0)[1], 0)
    pltpu.sync_copy(dst_hbm.at[wid], idx_v)
    plsc.subcore_barrier()

    def _body(j, _):
        pltpu.sync_copy(buf_v.at[pl.ds(0, CH)], acc_sh.at[idx_v.at[j]],
                        add=True)
        return 0

    lax.fori_loop(0, NCH, _body, 0)
    plsc.subcore_barrier()
    pltpu.sync_copy(acc_sh.at[pl.ds(drow0, RPT)],
                    out_hbm.at[pl.ds(dorow0, RPT)])


@functools.partial(
    pl.kernel,
    mesh=_mesh,
    out_type=jax.ShapeDtypeStruct((2 * N, D), jnp.float32),
    scratch_types=[
        pltpu.VMEM((2, CH), jnp.int32),
        pltpu.VMEM((2, CH), jnp.int32),
        pltpu.VMEM((2, CH), jnp.int32),
        pltpu.VMEM((2, CH), jnp.int32),
        pltpu.VMEM((CH, D), jnp.float32),
        pltpu.VMEM((CH, D), jnp.float32),
        pltpu.VMEM_SHARED((N, D), jnp.float32),
        pltpu.SemaphoreType.DMA,
        pltpu.SemaphoreType.DMA,
        pltpu.SemaphoreType.DMA,
        pltpu.SemaphoreType.DMA,
        pltpu.SemaphoreType.DMA,
        pltpu.SemaphoreType.DMA,
    ],
)
def _agg_sc(g_hbm, eidx_hbm, out_hbm, b0, b1, b2, b3, r0, r1,
            acc_sh, si0, si1, si2, si3, sr0, sr1):
    c = lax.axis_index("c")
    s = lax.axis_index("s")
    wid = c * 16 + s
    row0 = pl.multiple_of(s * RA, 8)
    orow0 = pl.multiple_of(c * N + s * RA, 8)
    bufs = (b0, b1, b2, b3)
    isems = (si0, si1, si2, si3)
    rows = (r0, r1)
    rsems = (sr0, sr1)

    def il(j, k):
        pltpu.async_copy(eidx_hbm.at[wid, j], bufs[k], isems[k])

    def wi(j, k):
        pltpu.make_async_copy(eidx_hbm.at[wid, j], bufs[k], isems[k]).wait()

    def ga(k, p):
        pltpu.async_copy(g_hbm.at[bufs[k].at[0]], rows[p], rsems[p])

    def wr(k, p):
        pltpu.make_async_copy(g_hbm.at[bufs[k].at[0]], rows[p],
                              rsems[p]).wait()

    def sc(k, p):
        pltpu.sync_copy(rows[p], acc_sh.at[bufs[k].at[1]], add=True)


    @pl.when(s < 15)
    def _():
        pltpu.sync_copy(g_hbm.at[pl.ds(row0, RA)], acc_sh.at[pl.ds(row0, RA)])

    @pl.when(s == 15)
    def _():
        pltpu.sync_copy(g_hbm.at[pl.ds(row0, RL)], acc_sh.at[pl.ds(row0, RL)])

    il(0, 0)
    il(1, 1)
    il(2, 2)
    il(3, 3)
    plsc.subcore_barrier()
    wi(0, 0)
    ga(0, 0)

    NB = NCH // 4

    def _body(bb, _):
        c0 = bb * 4
        last = bb >= NB - 1
        wi(c0 + 1, 1)
        ga(1, 1)
        wr(0, 0)
        sc(0, 0)

        @pl.when(~last)
        def _():
            il(c0 + 4, 0)

        wi(c0 + 2, 2)
        ga(2, 0)
        wr(1, 1)
        sc(1, 1)

        @pl.when(~last)
        def _():
            il(c0 + 5, 1)

        wi(c0 + 3, 3)
        ga(3, 1)
        wr(2, 0)
        sc(2, 0)

        @pl.when(~last)
        def _():
            il(c0 + 6, 2)
            wi(c0 + 4, 0)
            ga(0, 0)

        wr(3, 1)
        sc(3, 1)

        @pl.when(~last)
        def _():
            il(c0 + 7, 3)

        return 0

    lax.fori_loop(0, NB, _body, 0)
    plsc.subcore_barrier()

    @pl.when(s < 15)
    def _():
        pltpu.sync_copy(acc_sh.at[pl.ds(row0, RA)],
                        out_hbm.at[pl.ds(orow0, RA)])

    @pl.when(s == 15)
    def _():
        pltpu.sync_copy(acc_sh.at[pl.ds(row0, RL)],
                        out_hbm.at[pl.ds(orow0, RL)])


_BM = 2000


def _lin1_body(d0_ref, d1_ref, x_ref, w_ref, g_ref, dis_ref):
    deg = d0_ref[...] + d1_ref[...] + 1.0
    dis = lax.rsqrt(deg)
    hw = jnp.dot(x_ref[...], w_ref[...], preferred_element_type=jnp.float32)
    g_ref[...] = hw * dis
    dis_ref[...] = dis


def _lin2_body(p0_ref, p1_ref, g_ref, dis_ref, b_ref, w_ref, out_ref):
    dis = dis_ref[...]
    h = (p0_ref[...] + p1_ref[...] - g_ref[...]) * dis + b_ref[...]
    h = jnp.maximum(h, 0.0)
    out_ref[...] = jnp.dot(h, w_ref[...],
                           preferred_element_type=jnp.float32) * dis


def _pool_body(offs_ref, p0_ref, p1_ref, g_ref, dis_ref, b_ref, w_ref,
               bc_ref, out_ref, h_ref, pool_ref):
    h = (p0_ref[...] + p1_ref[...] - g_ref[...]) * dis_ref[...] + b_ref[...]
    h_ref[...] = jnp.maximum(h, 0.0)
    neg = jnp.float32(-jnp.inf)
    iota8 = lax.broadcasted_iota(jnp.int32, (8, 1), 0)

    def body(gi, _):
        s0 = offs_ref[gi]
        s1 = offs_ref[gi + 1]

        def inner(b, acc):
            ri = b * 8 + iota8
            m = (ri >= s0) & (ri < s1)
            rows = h_ref[pl.ds(b * 8, 8), :]
            return jnp.maximum(acc, jnp.where(m, rows, neg))

        acc = lax.fori_loop(s0 // 8, (s1 + 7) // 8,
                            inner, jnp.full((8, D), neg, jnp.float32))
        pool_ref[pl.ds(gi, 1), :] = jnp.max(acc, axis=0, keepdims=True)
        return 0

    lax.fori_loop(0, G, body, 0)
    out_ref[...] = jnp.dot(pool_ref[...], w_ref[...],
                           preferred_element_type=jnp.float32) + bc_ref[...]


def _row_blocked(width):
    return pl.BlockSpec((_BM, width), lambda i: (i, 0))


def _whole(shape):
    return pl.BlockSpec(shape, lambda *a: tuple(0 for _ in shape))


def kernel(x, edge_index, batch, W1, b1, W2, b2, W_iou, b_iou, W_iop, b_iop):
    src3 = edge_index[0].reshape(NW, NCH, CH)
    dst3 = edge_index[1].reshape(NW, NCH, CH)
    eidx = jnp.stack([src3, dst3], axis=2)

    degp = _deg_sc(dst3)
    deg0 = degp[0:N].reshape(N, 1)
    deg1 = degp[16 * RPT:16 * RPT + N].reshape(N, 1)

    grid = (N // _BM,)
    g1, dis = pl.pallas_call(
        _lin1_body,
        grid=grid,
        in_specs=[_row_blocked(1), _row_blocked(1), _row_blocked(D),
                  _whole((D, D))],
        out_specs=[_row_blocked(D), _row_blocked(1)],
        out_shape=[jax.ShapeDtypeStruct((N, D), jnp.float32),
                   jax.ShapeDtypeStruct((N, 1), jnp.float32)],
    )(deg0, deg1, x, W1)

    p = _agg_sc(g1, eidx)

    nb = N // _BM
    p_lo = pl.BlockSpec((_BM, D), lambda i: (i, 0))
    p_hi = pl.BlockSpec((_BM, D), lambda i: (i + nb, 0))
    g2 = pl.pallas_call(
        _lin2_body,
        grid=grid,
        in_specs=[p_lo, p_hi, _row_blocked(D),
                  _row_blocked(1), _whole((1, D)), _whole((D, D))],
        out_specs=_row_blocked(D),
        out_shape=jax.ShapeDtypeStruct((N, D), jnp.float32),
    )(p, p, g1, dis, b1.reshape(1, D), W2)

    p2 = _agg_sc(g2, eidx)

    w_cat = jnp.concatenate([W_iou, W_iop], axis=1)
    b_cat = jnp.concatenate([b_iou, b_iop]).reshape(1, 2)
    offs = jnp.searchsorted(batch, jnp.arange(G + 1, dtype=jnp.int32),
                            side="left").astype(jnp.int32)
    w_lo = pl.BlockSpec((N, D), lambda i: (0, 0))
    w_hi = pl.BlockSpec((N, D), lambda i: (1, 0))
    out = pl.pallas_call(
        _pool_body,
        grid=(1,),
        in_specs=[pl.BlockSpec(memory_space=pltpu.SMEM),
                  w_lo, w_hi, _whole((N, D)), _whole((N, 1)),
                  _whole((1, D)), _whole((D, 2)), _whole((1, 2))],
        out_specs=_whole((G, 2)),
        out_shape=jax.ShapeDtypeStruct((G, 2), jnp.float32),
        scratch_shapes=[pltpu.VMEM((N, D), jnp.float32),
                        pltpu.VMEM((G, D), jnp.float32)],
    )(offs, p2, p2, g2, dis, b2.reshape(1, D), w_cat, b_cat)
    return out
